# baseline (speedup 1.0000x reference)
import functools

import jax
import jax.numpy as jnp
from jax import lax
from jax.experimental import pallas as pl
from jax.experimental.pallas import tpu as pltpu
from jax.experimental.pallas import tpu_sc as plsc

_K = 4
_SC_BINS = 1024


def _votes_body(x_hbm, out_ref, bufs, sems):
    s = pl.program_id(0)
    nb = pl.num_programs(0)
    K, R, C = bufs.shape
    slot = lax.rem(s, K)

    @pl.when(s == 0)
    def _prologue():
        for k in range(K):
            pltpu.make_async_copy(
                x_hbm.at[pl.ds(k * R, R), :], bufs.at[k], sems.at[k]
            ).start()

    pltpu.make_async_copy(
        x_hbm.at[pl.ds(s * R, R), :], bufs.at[slot], sems.at[slot]
    ).wait()
    xb = bufs[slot]
    m = jnp.max(xb, axis=1, keepdims=True)
    iota = lax.broadcasted_iota(jnp.int32, (R, C), 1).astype(jnp.float32)
    cand = jnp.where(xb == m, iota, jnp.float32(C))
    vote = jnp.min(cand, axis=1)
    out_ref[0, 0, :] = vote.astype(jnp.int32)

    nxt = s + K

    @pl.when(nxt < nb)
    def _issue_next():
        pltpu.make_async_copy(
            x_hbm.at[pl.ds(nxt * R, R), :], bufs.at[slot], sems.at[slot]
        ).start()


def _tc_votes(x):
    N, C = x.shape
    R = 1000 if N % 1000 == 0 else N
    grid = N // R
    ring = min(_K, grid)
    votes = pl.pallas_call(
        _votes_body,
        grid=(grid,),
        in_specs=[pl.BlockSpec(memory_space=pltpu.HBM)],
        out_specs=pl.BlockSpec((1, 1, R), lambda i: (i, 0, 0)),
        out_shape=jax.ShapeDtypeStruct((grid, 1, R), jnp.int32),
        scratch_shapes=[
            pltpu.VMEM((ring, R, C), jnp.float32),
            pltpu.SemaphoreType.DMA((ring,)),
        ],
    )(x)
    return votes.reshape(N)


def _sc_hist_body(votes_hbm, out_hbm, votes_v, ones_v, hist_v, hist_sh):
    c = lax.axis_index("c")
    s = lax.axis_index("s")
    W = votes_v.shape[0]
    wid = s * 2 + c
    base = wid * W
    pltpu.sync_copy(votes_hbm.at[pl.ds(base, W)], votes_v)

    def _fill_ones(i, carry):
        ones_v[pl.ds(i * 16, 16)] = jnp.full((16,), 1, jnp.int32)
        return carry

    lax.fori_loop(0, W // 16, _fill_ones, 0)

    def _zero_hist(i, carry):
        hist_v[pl.ds(i * 16, 16)] = jnp.zeros((16,), jnp.int32)
        return carry

    lax.fori_loop(0, _SC_BINS // 16, _zero_hist, 0)

    @pl.when(s == 0)
    def _zero_shared():
        pltpu.sync_copy(hist_v, hist_sh)

    plsc.subcore_barrier()
    pltpu.sync_copy(ones_v, hist_sh.at[votes_v], add=True)
    plsc.subcore_barrier()

    @pl.when(s == 0)
    def _writeback():
        pltpu.sync_copy(hist_sh, hist_v)
        pltpu.sync_copy(hist_v, out_hbm.at[c])


def _sc_hist(votes_padded):
    NP = votes_padded.shape[0]
    W = NP // 32
    mesh = plsc.VectorSubcoreMesh(core_axis_name="c", subcore_axis_name="s")
    k = functools.partial(
        pl.kernel,
        out_type=jax.ShapeDtypeStruct((2, _SC_BINS), jnp.int32),
        mesh=mesh,
        scratch_types=[
            pltpu.VMEM((W,), jnp.int32),
            pltpu.VMEM((W,), jnp.int32),
            pltpu.VMEM((_SC_BINS,), jnp.int32),
            pltpu.VMEM_SHARED((_SC_BINS,), jnp.int32),
        ],
    )(_sc_hist_body)
    return k(votes_padded)


def _merge_body(part_ref, out_ref):
    C = out_ref.shape[1]
    counts = (part_ref[0, :] + part_ref[1, :]).astype(jnp.float32)
    iota = lax.iota(jnp.int32, _SC_BINS).astype(jnp.float32)
    counts = jnp.where(iota < jnp.float32(C), counts, jnp.float32(-1))
    cm = jnp.max(counts)
    cand = jnp.where(counts == cm, iota, jnp.float32(_SC_BINS))
    w = jnp.min(cand)
    iota_o = lax.iota(jnp.int32, C).astype(jnp.float32)
    out_ref[0, :] = (iota_o == w).astype(jnp.int32)


def _tc_merge(parts, C):
    out = pl.pallas_call(
        _merge_body,
        out_shape=jax.ShapeDtypeStruct((1, C), jnp.int32),
    )(parts)
    return out[0]


def kernel(x):
    N, C = x.shape
    votes = _tc_votes(x)
    pad = (-N) % (32 * 16)
    votes_padded = jnp.concatenate(
        [votes, jnp.full((pad,), C, jnp.int32)]) if pad else votes
    parts = _sc_hist(votes_padded)
    return _tc_merge(parts, C)
